# Initial kernel scaffold; baseline (speedup 1.0000x reference)
import jax
import jax.numpy as jnp
from jax.experimental import pallas as pl


def kernel(atoms, edge_s, edge_v, edge_index, batch, params):
    raise NotImplementedError("write your pallas kernel here")


# speedup vs baseline: 10.3863x; 10.3863x over previous
import functools

import jax
import jax.numpy as jnp
from jax import lax
from jax.experimental import pallas as pl
from jax.experimental.pallas import tpu as pltpu
from jax.experimental.pallas import tpu_sc as plsc

N = 10000
E = 160000
NG = 8
EPS = 1e-8

_NC, _NS = 2, 16
_NW = _NC * _NS
_CH = 128
_NCHUNK = E // _CH
_PER_W = _NCHUNK // _NW
_EXTRA = _NCHUNK - _PER_W * _NW

_TW = 256
_MW = 128
_NPAD = 10240
_RPT = _NPAD // _NS
_ZCH = 128

_BE = 2000
_BN = 2000


def _sigmoid(x):
    return 1.0 / (1.0 + jnp.exp(-x))


def _f32(shape):
    return jax.ShapeDtypeStruct(shape, jnp.float32)


@functools.lru_cache(maxsize=None)
def _build_gather():
    mesh = plsc.VectorSubcoreMesh(core_axis_name="c", subcore_axis_name="s")

    @functools.partial(
        pl.kernel,
        mesh=mesh,
        out_type=(_f32((E, _TW)), _f32((E, _TW))),
        scratch_types=[
            pltpu.VMEM((_CH,), jnp.int32),
            pltpu.VMEM((_CH,), jnp.int32),
            pltpu.VMEM((_CH, _TW), jnp.float32),
            pltpu.VMEM((_CH, _TW), jnp.float32),
            pltpu.SemaphoreType.DMA,
            pltpu.SemaphoreType.DMA,
        ],
    )
    def gather_k(u1, u2, src, dst, g1o, g2o, idx1, idx2, r1, r2, sem1, sem2):
        wid = lax.axis_index("s") * _NC + lax.axis_index("c")

        def do_chunk(g):
            off = g * _CH
            pltpu.sync_copy(src.at[pl.ds(off, _CH)], idx1)
            pltpu.sync_copy(dst.at[pl.ds(off, _CH)], idx2)
            c1 = pltpu.async_copy(u1.at[idx1], r1, sem1)
            c2 = pltpu.async_copy(u2.at[idx2], r2, sem2)
            c1.wait()
            c2.wait()
            pltpu.sync_copy(r1, g1o.at[pl.ds(off, _CH)])
            pltpu.sync_copy(r2, g2o.at[pl.ds(off, _CH)])

        def body(i, carry):
            do_chunk(wid * _PER_W + i)
            return carry

        lax.fori_loop(0, _PER_W, body, 0)

        @pl.when(wid < _EXTRA)
        def _():
            do_chunk(_NW * _PER_W + wid)

    return gather_k


@functools.lru_cache(maxsize=None)
def _build_scatter():
    mesh = plsc.VectorSubcoreMesh(core_axis_name="c", subcore_axis_name="s")

    @functools.partial(
        pl.kernel,
        mesh=mesh,
        out_type=(_f32((_NC * _NPAD, _MW)), _f32((_NC * _NPAD, _MW))),
        scratch_types=[
            pltpu.VMEM((_CH,), jnp.int32),
            pltpu.VMEM((_CH, _MW), jnp.float32),
            pltpu.VMEM((_ZCH, _MW), jnp.float32),
            pltpu.VMEM_SHARED((_NPAD, _MW), jnp.float32),
        ],
    )
    def scatter_k(m1, m2, dst, zrow, out1, out2, idx, rows, zbuf, acc):
        cid = lax.axis_index("c")
        sid = lax.axis_index("s")
        wid = sid * _NC + cid
        pltpu.sync_copy(zrow, zbuf)

        def one_pass(m, out):
            for k in range(_RPT // _ZCH):
                pltpu.sync_copy(zbuf, acc.at[pl.ds(sid * _RPT + k * _ZCH, _ZCH)])
            plsc.subcore_barrier()

            def do_chunk(g):
                off = g * _CH
                pltpu.sync_copy(dst.at[pl.ds(off, _CH)], idx)
                pltpu.sync_copy(m.at[pl.ds(off, _CH)], rows)
                pltpu.sync_copy(rows, acc.at[idx], add=True)

            def body(i, carry):
                do_chunk(wid * _PER_W + i)
                return carry

            lax.fori_loop(0, _PER_W, body, 0)

            @pl.when(wid < _EXTRA)
            def _():
                do_chunk(_NW * _PER_W + wid)

            plsc.subcore_barrier()
            for k in range(_RPT // _ZCH):
                r0 = sid * _RPT + k * _ZCH
                pltpu.sync_copy(acc.at[pl.ds(r0, _ZCH)], rows)
                pltpu.sync_copy(rows, out.at[pl.ds(cid * _NPAD + r0, _ZCH)])

        one_pass(m1, out1)
        plsc.subcore_barrier()
        one_pass(m2, out2)

    return scatter_k


def _gather_pairs(u1, u2, src, dst):
    return _build_gather()(u1, u2, src, dst)


def _scatter_add(m1, m2, dst, zrow):
    return _build_scatter()(m1, m2, dst, zrow)


def _const_spec(shape):
    return pl.BlockSpec(shape, lambda i: tuple(0 for _ in shape))


def _node_init_tc(atoms2d, t9):
    def body(a_ref, t_ref, o_ref):
        a = a_ref[...]
        oh = (a == lax.broadcasted_iota(jnp.int32, (_BN, 16), 1)).astype(jnp.float32)
        o_ref[...] = jnp.dot(oh, t_ref[...], preferred_element_type=jnp.float32,
                             precision=lax.Precision.HIGHEST)

    t9p = jnp.zeros((16, 100), jnp.float32).at[:9].set(t9)
    return pl.pallas_call(
        body,
        grid=(N // _BN,),
        in_specs=[
            pl.BlockSpec((_BN, 1), lambda i: (i, 0)),
            _const_spec((16, 100)),
        ],
        out_specs=pl.BlockSpec((_BN, 100), lambda i: (i, 0)),
        out_shape=_f32((N, 100)),
    )(atoms2d, t9p)


def _edge_init_tc(edge_s, ev3, w):
    def body(es_ref, ev_ref, lng, lnb, wsw, wsb, wsvw, wsvb, whwv, eso_ref, evo_ref):
        ev = ev_ref[...]
        vn2 = jnp.maximum(jnp.sum(ev * ev, axis=1, keepdims=True), EPS)
        evn = ev / jnp.sqrt(vn2)
        s = es_ref[...]
        mu = jnp.mean(s, axis=1, keepdims=True)
        var = jnp.mean(jnp.square(s - mu), axis=1, keepdims=True)
        s = (s - mu) / jnp.sqrt(var + 1e-5) * lng[...] + lnb[...]
        wh = whwv[0, 0]
        wv = whwv[0, 1]
        vh = wh * evn
        vnorm = jnp.sqrt(jnp.maximum(jnp.sum(vh * vh, axis=1, keepdims=True), EPS))
        sin = jnp.concatenate([s, vnorm], axis=1)
        es2 = jnp.dot(sin, wsw[...], preferred_element_type=jnp.float32) + wsb[...]
        gate = _sigmoid(jnp.dot(es2, wsvw[...], preferred_element_type=jnp.float32) + wsvb[...])
        evo_ref[...] = wv * vh * gate
        eso_ref[...] = es2

    g = w['We_gvp']
    whwv = jnp.array([[g['wh'][0, 0], g['wv'][0, 0]]], jnp.float32)
    return pl.pallas_call(
        body,
        grid=(E // _BE,),
        in_specs=[
            pl.BlockSpec((_BE, 16), lambda i: (i, 0)),
            pl.BlockSpec((_BE, 3), lambda i: (i, 0)),
            _const_spec((1, 16)), _const_spec((1, 16)),
            _const_spec((17, 32)), _const_spec((1, 32)),
            _const_spec((32, 1)), _const_spec((1, 1)),
            _const_spec((1, 2)),
        ],
        out_specs=[
            pl.BlockSpec((_BE, 32), lambda i: (i, 0)),
            pl.BlockSpec((_BE, 3), lambda i: (i, 0)),
        ],
        out_shape=[_f32((E, 32)), _f32((E, 3))],
    )(edge_s, ev3,
      w['We_ln']['g'][None], w['We_ln']['b'][None],
      g['ws']['w'].T, g['ws']['b'][None],
      g['wsv']['w'].T, g['wsv']['b'][None],
      whwv)


def _precompute_tc(s, v, wsrc, wdst, whsrc, whdst):
    def body(s_ref, v_ref, w1, w2, h1, h2, u1_ref, u2_ref):
        sv = s_ref[...]
        vv = v_ref[...]
        z4 = jnp.zeros((_BN, 4), jnp.float32)
        z7 = jnp.zeros((_BN, 7), jnp.float32)
        z39 = jnp.zeros((_BN, 39), jnp.float32)

        def table(wt, ht):
            a = jnp.dot(sv, wt, preferred_element_type=jnp.float32)
            px = jnp.dot(vv[:, 0:16], ht, preferred_element_type=jnp.float32)
            py = jnp.dot(vv[:, 16:32], ht, preferred_element_type=jnp.float32)
            pz = jnp.dot(vv[:, 32:48], ht, preferred_element_type=jnp.float32)
            return jnp.concatenate([a, z4, px, z7, py, z7, pz, z39], axis=1)

        u1_ref[...] = table(w1[...], h1[...])
        u2_ref[...] = table(w2[...], h2[...])

    return pl.pallas_call(
        body,
        grid=(N // _BN,),
        in_specs=[
            pl.BlockSpec((_BN, 100), lambda i: (i, 0)),
            pl.BlockSpec((_BN, 48), lambda i: (i, 0)),
            _const_spec((100, 100)), _const_spec((100, 100)),
            _const_spec((16, 33)), _const_spec((16, 33)),
        ],
        out_specs=[
            pl.BlockSpec((_BN, _TW), lambda i: (i, 0)),
            pl.BlockSpec((_BN, _TW), lambda i: (i, 0)),
        ],
        out_shape=[_f32((N, _TW)), _f32((N, _TW))],
    )(s, v, wsrc, wdst, whsrc, whdst)


def _edge_tc(g1, g2, es, ev3, ew):
    def body(g1_ref, g2_ref, es_ref, ev_ref,
             whev, wes, wvn, b0, wsv0, bsv0, wv0,
             wh1, ws1a, ws1b, b1, wsv1, bsv1, wv1,
             wh2, ws2a, ws2b, b2, wsv2, bsv2, wv2,
             m1_ref, m2_ref):
        G1 = g1_ref[...]
        G2 = g2_ref[...]
        e3 = ev_ref[...]
        hv = whev[...]
        A = G1[:, 0:100] + G2[:, 0:100]
        vhx = G1[:, 104:137] + G2[:, 104:137] + e3[:, 0:1] * hv
        vhy = G1[:, 144:177] + G2[:, 144:177] + e3[:, 1:2] * hv
        vhz = G1[:, 184:217] + G2[:, 184:217] + e3[:, 2:3] * hv
        vn = jnp.sqrt(jnp.maximum(vhx * vhx + vhy * vhy + vhz * vhz, EPS))
        s0 = (A
              + jnp.dot(es_ref[...], wes[...], preferred_element_type=jnp.float32)
              + jnp.dot(vn, wvn[...], preferred_element_type=jnp.float32)
              + b0[...])
        gate = _sigmoid(jnp.dot(s0, wsv0[...], preferred_element_type=jnp.float32) + bsv0[...])
        mx = jnp.dot(vhx, wv0[...], preferred_element_type=jnp.float32) * gate
        my = jnp.dot(vhy, wv0[...], preferred_element_type=jnp.float32) * gate
        mz = jnp.dot(vhz, wv0[...], preferred_element_type=jnp.float32) * gate
        ms = jnp.maximum(s0, 0.0)

        for (wh, wsa, wsb, b, wsv, bsv, wv, act) in (
                (wh1, ws1a, ws1b, b1, wsv1, bsv1, wv1, True),
                (wh2, ws2a, ws2b, b2, wsv2, bsv2, wv2, False)):
            vhx = jnp.dot(mx, wh[...], preferred_element_type=jnp.float32)
            vhy = jnp.dot(my, wh[...], preferred_element_type=jnp.float32)
            vhz = jnp.dot(mz, wh[...], preferred_element_type=jnp.float32)
            vn = jnp.sqrt(jnp.maximum(vhx * vhx + vhy * vhy + vhz * vhz, EPS))
            s0 = (jnp.dot(ms, wsa[...], preferred_element_type=jnp.float32)
                  + jnp.dot(vn, wsb[...], preferred_element_type=jnp.float32)
                  + b[...])
            gate = _sigmoid(jnp.dot(s0, wsv[...], preferred_element_type=jnp.float32) + bsv[...])
            mx = jnp.dot(vhx, wv[...], preferred_element_type=jnp.float32) * gate
            my = jnp.dot(vhy, wv[...], preferred_element_type=jnp.float32) * gate
            mz = jnp.dot(vhz, wv[...], preferred_element_type=jnp.float32) * gate
            ms = jnp.maximum(s0, 0.0) if act else s0

        m1_ref[...] = jnp.concatenate([
            ms, jnp.ones((_BE, 1), jnp.float32), jnp.zeros((_BE, 27), jnp.float32),
        ], axis=1)
        m2_ref[...] = jnp.concatenate([
            mx, my, mz, jnp.zeros((_BE, 80), jnp.float32),
        ], axis=1)

    return pl.pallas_call(
        body,
        grid=(E // _BE,),
        in_specs=[
            pl.BlockSpec((_BE, _TW), lambda i: (i, 0)),
            pl.BlockSpec((_BE, _TW), lambda i: (i, 0)),
            pl.BlockSpec((_BE, 32), lambda i: (i, 0)),
            pl.BlockSpec((_BE, 3), lambda i: (i, 0)),
            _const_spec((1, 33)),
            _const_spec((32, 100)), _const_spec((33, 100)), _const_spec((1, 100)),
            _const_spec((100, 16)), _const_spec((1, 16)), _const_spec((33, 16)),
            _const_spec((16, 16)), _const_spec((100, 100)), _const_spec((16, 100)),
            _const_spec((1, 100)), _const_spec((100, 16)), _const_spec((1, 16)),
            _const_spec((16, 16)),
            _const_spec((16, 16)), _const_spec((100, 100)), _const_spec((16, 100)),
            _const_spec((1, 100)), _const_spec((100, 16)), _const_spec((1, 16)),
            _const_spec((16, 16)),
        ],
        out_specs=[
            pl.BlockSpec((_BE, _MW), lambda i: (i, 0)),
            pl.BlockSpec((_BE, _MW), lambda i: (i, 0)),
        ],
        out_shape=[_f32((E, _MW)), _f32((E, _MW))],
    )(g1, g2, es, ev3, *ew)


def _ln_sv_block(g, b, s, vx, vy, vz):
    vn2 = jnp.maximum(vx * vx + vy * vy + vz * vz, EPS)
    denom = jnp.sqrt(jnp.mean(vn2, axis=1, keepdims=True))
    mu = jnp.mean(s, axis=1, keepdims=True)
    var = jnp.mean(jnp.square(s - mu), axis=1, keepdims=True)
    s = (s - mu) / jnp.sqrt(var + 1e-5) * g + b
    return s, vx / denom, vy / denom, vz / denom


def _node_tc(p1, p2, s, v, nw):
    def body(p1_ref, p2_ref, s_ref, v_ref,
             n0g, n0b, f0wh, f0wsa, f0wsb, f0b, f0wsv, f0bsv, f0wv,
             f1wh, f1wsa, f1wsb, f1b, f1wsv, f1bsv, f1wv, n1g, n1b,
             so_ref, vo_ref):
        pa = p1_ref[0] + p1_ref[1]
        pb = p2_ref[0] + p2_ref[1]
        inv = 1.0 / jnp.maximum(pa[:, 100:101], 1.0)
        s = s_ref[...] + pa[:, 0:100] * inv
        vv = v_ref[...]
        vx = vv[:, 0:16] + pb[:, 0:16] * inv
        vy = vv[:, 16:32] + pb[:, 16:32] * inv
        vz = vv[:, 32:48] + pb[:, 32:48] * inv
        s, vx, vy, vz = _ln_sv_block(n0g[...], n0b[...], s, vx, vy, vz)

        hx = jnp.dot(vx, f0wh[...], preferred_element_type=jnp.float32)
        hy = jnp.dot(vy, f0wh[...], preferred_element_type=jnp.float32)
        hz = jnp.dot(vz, f0wh[...], preferred_element_type=jnp.float32)
        vn = jnp.sqrt(jnp.maximum(hx * hx + hy * hy + hz * hz, EPS))
        hs = (jnp.dot(s, f0wsa[...], preferred_element_type=jnp.float32)
              + jnp.dot(vn, f0wsb[...], preferred_element_type=jnp.float32)
              + f0b[...])
        gate = _sigmoid(jnp.dot(hs, f0wsv[...], preferred_element_type=jnp.float32) + f0bsv[...])
        hx = jnp.dot(hx, f0wv[...], preferred_element_type=jnp.float32) * gate
        hy = jnp.dot(hy, f0wv[...], preferred_element_type=jnp.float32) * gate
        hz = jnp.dot(hz, f0wv[...], preferred_element_type=jnp.float32) * gate
        hs = jnp.maximum(hs, 0.0)

        dx = jnp.dot(hx, f1wh[...], preferred_element_type=jnp.float32)
        dy = jnp.dot(hy, f1wh[...], preferred_element_type=jnp.float32)
        dz = jnp.dot(hz, f1wh[...], preferred_element_type=jnp.float32)
        vn = jnp.sqrt(jnp.maximum(dx * dx + dy * dy + dz * dz, EPS))
        ds = (jnp.dot(hs, f1wsa[...], preferred_element_type=jnp.float32)
              + jnp.dot(vn, f1wsb[...], preferred_element_type=jnp.float32)
              + f1b[...])
        gate = _sigmoid(jnp.dot(ds, f1wsv[...], preferred_element_type=jnp.float32) + f1bsv[...])
        dx = jnp.dot(dx, f1wv[...], preferred_element_type=jnp.float32) * gate
        dy = jnp.dot(dy, f1wv[...], preferred_element_type=jnp.float32) * gate
        dz = jnp.dot(dz, f1wv[...], preferred_element_type=jnp.float32) * gate

        s = s + ds
        vx, vy, vz = vx + dx, vy + dy, vz + dz
        s, vx, vy, vz = _ln_sv_block(n1g[...], n1b[...], s, vx, vy, vz)
        so_ref[...] = s
        vo_ref[...] = jnp.concatenate([vx, vy, vz], axis=1)

    return pl.pallas_call(
        body,
        grid=(N // _BN,),
        in_specs=[
            pl.BlockSpec((2, _BN, _MW), lambda i: (0, i, 0)),
            pl.BlockSpec((2, _BN, _MW), lambda i: (0, i, 0)),
            pl.BlockSpec((_BN, 100), lambda i: (i, 0)),
            pl.BlockSpec((_BN, 48), lambda i: (i, 0)),
            _const_spec((1, 100)), _const_spec((1, 100)),
            _const_spec((16, 32)), _const_spec((100, 400)), _const_spec((32, 400)),
            _const_spec((1, 400)), _const_spec((400, 32)), _const_spec((1, 32)),
            _const_spec((32, 32)),
            _const_spec((32, 32)), _const_spec((400, 100)), _const_spec((32, 100)),
            _const_spec((1, 100)), _const_spec((100, 16)), _const_spec((1, 16)),
            _const_spec((32, 16)),
            _const_spec((1, 100)), _const_spec((1, 100)),
        ],
        out_specs=[
            pl.BlockSpec((_BN, 100), lambda i: (i, 0)),
            pl.BlockSpec((_BN, 48), lambda i: (i, 0)),
        ],
        out_shape=[_f32((N, 100)), _f32((N, 48))],
    )(p1, p2, s, v, *nw)


def _readout_tc(s, v, batch2d, w):
    def body(s_ref, v_ref, b_ref, lng, lnb, wh, wsa, wsb, bb, acc_ref):
        @pl.when(pl.program_id(0) == 0)
        def _():
            acc_ref[...] = jnp.zeros((NG, 104), jnp.float32)

        s = s_ref[...]
        vv = v_ref[...]
        vx, vy, vz = vv[:, 0:16], vv[:, 16:32], vv[:, 32:48]
        s, vx, vy, vz = _ln_sv_block(lng[...], lnb[...], s, vx, vy, vz)
        hx = jnp.dot(vx, wh[...], preferred_element_type=jnp.float32)
        hy = jnp.dot(vy, wh[...], preferred_element_type=jnp.float32)
        hz = jnp.dot(vz, wh[...], preferred_element_type=jnp.float32)
        vn = jnp.sqrt(jnp.maximum(hx * hx + hy * hy + hz * hz, EPS))
        out = jnp.maximum(
            jnp.dot(s, wsa[...], preferred_element_type=jnp.float32)
            + jnp.dot(vn, wsb[...], preferred_element_type=jnp.float32)
            + bb[...], 0.0)
        oh = (b_ref[...] == lax.broadcasted_iota(jnp.int32, (_BN, NG), 1)).astype(jnp.float32)
        ext = jnp.concatenate([
            out, jnp.ones((_BN, 1), jnp.float32), jnp.zeros((_BN, 3), jnp.float32),
        ], axis=1)
        acc_ref[...] += jnp.dot(oh.T, ext, preferred_element_type=jnp.float32,
                                precision=lax.Precision.HIGHEST)

    g = w['Wout_gvp']
    return pl.pallas_call(
        body,
        grid=(N // _BN,),
        in_specs=[
            pl.BlockSpec((_BN, 100), lambda i: (i, 0)),
            pl.BlockSpec((_BN, 48), lambda i: (i, 0)),
            pl.BlockSpec((_BN, 1), lambda i: (i, 0)),
            _const_spec((1, 100)), _const_spec((1, 100)),
            _const_spec((16, 16)),
            _const_spec((100, 100)), _const_spec((16, 100)), _const_spec((1, 100)),
        ],
        out_specs=pl.BlockSpec((NG, 104), lambda i: (0, 0)),
        out_shape=_f32((NG, 104)),
    )(s, v, batch2d,
      w['Wout_ln']['g'][None], w['Wout_ln']['b'][None],
      g['wh'].T, g['ws']['w'][:, :100].T, g['ws']['w'][:, 100:].T, g['ws']['b'][None])


def _head_tc(acc, w):
    def body(a_ref, d0w, d0b, d1w, d1b, o_ref):
        a = a_ref[...]
        cnt = jnp.maximum(a[:, 100:101], 1.0)
        gmean = a[:, 0:100] / cnt
        h = jnp.maximum(
            jnp.dot(gmean, d0w[...], preferred_element_type=jnp.float32) + d0b[...], 0.0)
        o_ref[...] = jnp.sum(h * d1w[...], axis=1, keepdims=True) + d1b[...]

    return pl.pallas_call(
        body,
        grid=(1,),
        in_specs=[
            _const_spec((NG, 104)),
            _const_spec((100, 200)), _const_spec((1, 200)),
            _const_spec((1, 200)), _const_spec((1, 1)),
        ],
        out_specs=pl.BlockSpec((NG, 1), lambda i: (0, 0)),
        out_shape=_f32((NG, 1)),
    )(acc, w['d0']['w'].T, w['d0']['b'][None],
      w['d1']['w'], w['d1']['b'][None, :])


def _ln_rows(p, s):
    mu = jnp.mean(s, axis=-1, keepdims=True)
    var = jnp.mean(jnp.square(s - mu), axis=-1, keepdims=True)
    return (s - mu) / jnp.sqrt(var + 1e-5) * p['g'] + p['b']


def kernel(atoms, edge_s, edge_v, edge_index, batch, params):
    src = edge_index[0]
    dst = edge_index[1]
    ev3 = edge_v[:, 0, :]

    t9 = (_ln_rows(params['Wv_ln'], params['embed']) @ params['Wv_ws']['w'].T
          + params['Wv_ws']['b'])
    s = _node_init_tc(atoms[:, None].astype(jnp.int32), t9)
    v = jnp.zeros((N, 48), jnp.float32)

    es, evw = _edge_init_tc(edge_s, ev3, params)

    zrow = jnp.zeros((_ZCH, _MW), jnp.float32)

    for lp in params['layers']:
        g0, g1p, g2p = lp['g0'], lp['g1'], lp['g2']
        ws0 = g0['ws']['w']
        wh0 = g0['wh']
        u1, u2 = _precompute_tc(
            s, v,
            ws0[:, :100].T, ws0[:, 132:232].T,
            wh0[:, :16].T, wh0[:, 17:].T)
        gg1, gg2 = _gather_pairs(u1, u2, src, dst)
        ew = (
            wh0[:, 16][None],
            ws0[:, 100:132].T, ws0[:, 232:].T, g0['ws']['b'][None],
            g0['wsv']['w'].T, g0['wsv']['b'][None], g0['wv'].T,
            g1p['wh'].T, g1p['ws']['w'][:, :100].T, g1p['ws']['w'][:, 100:].T,
            g1p['ws']['b'][None], g1p['wsv']['w'].T, g1p['wsv']['b'][None], g1p['wv'].T,
            g2p['wh'].T, g2p['ws']['w'][:, :100].T, g2p['ws']['w'][:, 100:].T,
            g2p['ws']['b'][None], g2p['wsv']['w'].T, g2p['wsv']['b'][None], g2p['wv'].T,
        )
        m1, m2 = _edge_tc(gg1, gg2, es, evw, ew)
        p1, p2 = _scatter_add(m1, m2, dst, zrow)
        p1 = p1.reshape(_NC, _NPAD, _MW)
        p2 = p2.reshape(_NC, _NPAD, _MW)
        f0, f1 = lp['ff0'], lp['ff1']
        nw = (
            lp['norm0']['g'][None], lp['norm0']['b'][None],
            f0['wh'].T, f0['ws']['w'][:, :100].T, f0['ws']['w'][:, 100:].T,
            f0['ws']['b'][None], f0['wsv']['w'].T, f0['wsv']['b'][None], f0['wv'].T,
            f1['wh'].T, f1['ws']['w'][:, :400].T, f1['ws']['w'][:, 400:].T,
            f1['ws']['b'][None], f1['wsv']['w'].T, f1['wsv']['b'][None], f1['wv'].T,
            lp['norm1']['g'][None], lp['norm1']['b'][None],
        )
        s, v = _node_tc(p1, p2, s, v, nw)

    acc = _readout_tc(s, v, batch[:, None].astype(jnp.int32), params)
    out = _head_tc(acc, params)
    return out[:, 0]
